# baseline (speedup 1.0000x reference)
import functools

import jax
import jax.numpy as jnp
from jax import lax
from jax.experimental import pallas as pl
from jax.experimental.pallas import tpu as pltpu, tpu_sc as plsc

D_MODEL = 64
BATCH = 16384
L = 16
CHUNK = 128


def _embed_call(tokens_i32, W2):
    info = plsc.get_sparse_core_info()
    nw = info.num_cores * info.num_subcores
    b_per_w = BATCH // nw
    n_chunks = b_per_w // CHUNK
    mesh = plsc.VectorSubcoreMesh(core_axis_name="c", subcore_axis_name="s")

    @functools.partial(
        pl.kernel,
        mesh=mesh,
        out_type=jax.ShapeDtypeStruct((BATCH, D_MODEL), jnp.float32),
        scratch_types=[
            pltpu.VMEM((b_per_w,), jnp.int32),
            pltpu.VMEM((b_per_w,), jnp.int32),
            pltpu.VMEM((CHUNK, 2 * D_MODEL), jnp.float32),
            pltpu.VMEM((b_per_w, D_MODEL), jnp.float32),
            pltpu.SemaphoreType.DMA,
        ],
        compiler_params=pltpu.CompilerParams(needs_layout_passes=False),
    )
    def k(idx_hbm, table_hbm, out_hbm, idx_v, half_v, rows2_v, out_v, sem):
        wid = lax.axis_index("s") * info.num_cores + lax.axis_index("c")
        base = wid * b_per_w
        pltpu.sync_copy(idx_hbm.at[pl.ds(base, b_per_w)], idx_v)

        lanes = lax.iota(jnp.int32, L)

        def to_half(g, _):
            tv = idx_v[pl.ds(g * L, L)]
            half_v[pl.ds(g * L, L)] = lax.shift_right_logical(tv, 1)
            return ()

        lax.fori_loop(0, b_per_w // L, to_half, (), unroll=False)

        def chunk_body(c, _):
            pltpu.async_copy(
                table_hbm.at[half_v.at[pl.ds(c * CHUNK, CHUNK)]], rows2_v, sem
            ).wait()

            def select(g, _):
                tv = idx_v[pl.ds(c * CHUNK + g * L, L)]
                rowids = lanes + g * L
                src_col = (tv & 1) * D_MODEL
                for j in range(D_MODEL):
                    vals = plsc.load_gather(rows2_v, [rowids, src_col + j])
                    plsc.store_scatter(
                        out_v,
                        [c * CHUNK + rowids, jnp.full((L,), j, jnp.int32)],
                        vals,
                    )
                return ()

            lax.fori_loop(0, CHUNK // L, select, (), unroll=False)
            return ()

        lax.fori_loop(0, n_chunks, chunk_body, (), unroll=False)

        pltpu.sync_copy(out_v, out_hbm.at[pl.ds(base, b_per_w)])

    return k(tokens_i32, W2)


def kernel(tokens, W_E):
    W2 = W_E.reshape(W_E.shape[0] // 2, 2 * W_E.shape[1])
    return _embed_call(tokens.astype(jnp.int32), W2)


# speedup vs baseline: 1.0586x; 1.0586x over previous
import functools

import jax
import jax.numpy as jnp
from jax import lax
from jax.experimental import pallas as pl
from jax.experimental.pallas import tpu as pltpu, tpu_sc as plsc

D_MODEL = 64
BATCH = 16384


def _embed_call(tokens_i32, W_E):
    info = plsc.get_sparse_core_info()
    nw = info.num_cores * info.num_subcores
    b_per_w = BATCH // nw
    mesh = plsc.VectorSubcoreMesh(core_axis_name="c", subcore_axis_name="s")

    @functools.partial(
        pl.kernel,
        mesh=mesh,
        out_type=jax.ShapeDtypeStruct((BATCH, D_MODEL), jnp.float32),
        scratch_types=[
            pltpu.VMEM((b_per_w,), jnp.int32),
            pltpu.VMEM((b_per_w, D_MODEL), jnp.float32),
            pltpu.SemaphoreType.DMA,
        ],
        compiler_params=pltpu.CompilerParams(
            use_tc_tiling_on_sc=False, needs_layout_passes=False
        ),
    )
    def k(idx_hbm, table_hbm, out_hbm, idx_v, rows_v, sem):
        wid = lax.axis_index("s") * info.num_cores + lax.axis_index("c")
        base = wid * b_per_w
        pltpu.sync_copy(idx_hbm.at[pl.ds(base, b_per_w)], idx_v)
        pltpu.async_copy(table_hbm.at[idx_v], rows_v, sem).wait()
        pltpu.sync_copy(rows_v, out_hbm.at[pl.ds(base, b_per_w)])

    return k(tokens_i32, W_E)


def kernel(tokens, W_E):
    return _embed_call(tokens.astype(jnp.int32), W_E)
